# Initial kernel scaffold; baseline (speedup 1.0000x reference)
import jax
import jax.numpy as jnp
from jax.experimental import pallas as pl


def kernel(x, edge_index, batch, W1, b1, W2, b2, W3, b3, Wl, bl):
    raise NotImplementedError("write your pallas kernel here")


# speedup vs baseline: 12.7892x; 12.7892x over previous
import functools

import jax
import jax.numpy as jnp
from jax import lax
from jax.experimental import pallas as pl
from jax.experimental.pallas import tpu as pltpu
from jax.experimental.pallas import tpu_sc as plsc

N = 10000
E = 320000
D = 128
H = 64
G = 128
O = 2

NC = 2
NS = 16
NW = NC * NS
CHUNK = 128
K = 80
EPW = K * CHUNK
E_PAD = NW * EPW
NP = 10112
RPT = NP // NS
DUMP = 10104
DEG_W = 8


def _deg_body(dst3, zrows, ones_h, out, di_all, ones_v, stage, acc):
    c = lax.axis_index("c")
    s = lax.axis_index("s")
    wid = c * NS + s
    r0 = s * RPT
    pltpu.sync_copy(zrows.at[pl.ds(r0, RPT)], stage)
    pltpu.sync_copy(stage, acc.at[pl.ds(r0, RPT)])
    pltpu.sync_copy(ones_h, ones_v)
    pltpu.sync_copy(dst3.at[wid], di_all)
    plsc.subcore_barrier()

    def body(g, carry):
        pltpu.sync_copy(ones_v, acc.at[di_all.at[g]], add=True)
        return carry

    lax.fori_loop(0, K, body, 0)
    plsc.subcore_barrier()
    pltpu.sync_copy(acc.at[pl.ds(r0, RPT)], stage)
    pltpu.sync_copy(stage, out.at[c, pl.ds(r0, RPT)])


def _agg_body(y, src3, dst3, zrows, out, si_all, di_all, rows, stage, acc,
              sem0, sem1):
    c = lax.axis_index("c")
    s = lax.axis_index("s")
    wid = c * NS + s
    r0 = s * RPT
    pltpu.sync_copy(zrows.at[pl.ds(r0, RPT)], stage)
    pltpu.sync_copy(stage, acc.at[pl.ds(r0, RPT)])
    pltpu.sync_copy(src3.at[wid], si_all)
    pltpu.sync_copy(dst3.at[wid], di_all)
    plsc.subcore_barrier()

    sems = (sem0, sem1)
    pltpu.async_copy(y.at[si_all.at[0]], rows.at[0], sem0)

    def body(i, carry):
        for b in range(2):
            g = 2 * i + b
            pltpu.make_async_copy(y.at[si_all.at[g]], rows.at[b], sems[b]).wait()

            @pl.when(g + 1 < K)
            def _():
                pltpu.async_copy(
                    y.at[si_all.at[g + 1]], rows.at[1 - b], sems[1 - b]
                )

            pltpu.sync_copy(rows.at[b], acc.at[di_all.at[g]], add=True)
        return carry

    lax.fori_loop(0, K // 2, body, 0)
    plsc.subcore_barrier()
    pltpu.sync_copy(acc.at[pl.ds(r0, RPT)], stage)
    pltpu.sync_copy(stage, out.at[c, pl.ds(r0, RPT)])


@functools.cache
def _sc_kernels():
    mesh = plsc.VectorSubcoreMesh(
        core_axis_name="c", subcore_axis_name="s",
        num_cores=NC, num_subcores=NS,
    )
    params = pltpu.CompilerParams(use_tc_tiling_on_sc=False)
    deg = pl.kernel(
        _deg_body,
        out_type=jax.ShapeDtypeStruct((NC, NP, DEG_W), jnp.float32),
        mesh=mesh,
        compiler_params=params,
        scratch_types=[
            pltpu.VMEM((K, CHUNK), jnp.int32),
            pltpu.VMEM((CHUNK, DEG_W), jnp.float32),
            pltpu.VMEM((RPT, DEG_W), jnp.float32),
            pltpu.VMEM_SHARED((NP, DEG_W), jnp.float32),
        ],
    )
    agg = pl.kernel(
        _agg_body,
        out_type=jax.ShapeDtypeStruct((NC, NP, H), jnp.float32),
        mesh=mesh,
        compiler_params=params,
        scratch_types=[
            pltpu.VMEM((K, CHUNK), jnp.int32),
            pltpu.VMEM((K, CHUNK), jnp.int32),
            pltpu.VMEM((2, CHUNK, H), jnp.float32),
            pltpu.VMEM((RPT, H), jnp.float32),
            pltpu.VMEM_SHARED((NP, H), jnp.float32),
            pltpu.SemaphoreType.DMA,
            pltpu.SemaphoreType.DMA,
        ],
    )
    return deg, agg


def _tc_first_body(deg8_ref, x_ref, w_ref, y_ref, dinv_ref):
    deg = deg8_ref[0, :, 0:1] + deg8_ref[1, :, 0:1] + 1.0
    dinv = lax.rsqrt(deg)
    xw = jnp.dot(x_ref[...], w_ref[...], preferred_element_type=jnp.float32)
    y_ref[...] = xw * dinv
    dinv_ref[...] = dinv


_tc_first = pl.pallas_call(
    _tc_first_body,
    out_shape=(
        jax.ShapeDtypeStruct((NP, H), jnp.float32),
        jax.ShapeDtypeStruct((NP, 1), jnp.float32),
    ),
)


def _tc_mid_body(agg_ref, y_ref, dinv_ref, b_ref, w_ref, out_ref):
    dinv = dinv_ref[...]
    tot = agg_ref[0] + agg_ref[1] + y_ref[...]
    h = jnp.maximum(tot * dinv + b_ref[...], 0.0)
    out_ref[...] = (
        jnp.dot(h, w_ref[...], preferred_element_type=jnp.float32) * dinv
    )


_tc_mid = pl.pallas_call(
    _tc_mid_body,
    out_shape=jax.ShapeDtypeStruct((NP, H), jnp.float32),
)


def _tc_final_body(agg_ref, y_ref, dinv_ref, b_ref, batch_ref, wl_ref, bl_ref,
                   out_ref):
    dinv = dinv_ref[...]
    tot = agg_ref[0] + agg_ref[1] + y_ref[...]
    h = jnp.maximum(tot * dinv + b_ref[...], 0.0)
    gid = lax.broadcasted_iota(jnp.int32, (NP, G), 1)
    oh = (batch_ref[...] == gid).astype(jnp.float32)
    dn = (((0,), (0,)), ((), ()))
    sums = lax.dot_general(oh, h, dn, preferred_element_type=jnp.float32)
    counts = lax.dot_general(
        oh, jnp.ones((NP, 1), jnp.float32), dn,
        preferred_element_type=jnp.float32,
    )
    pooled = sums / jnp.maximum(counts, 1.0)
    out_ref[...] = (
        jnp.dot(pooled, wl_ref[...], preferred_element_type=jnp.float32)
        + bl_ref[...]
    )


_tc_final = pl.pallas_call(
    _tc_final_body,
    out_shape=jax.ShapeDtypeStruct((G, O), jnp.float32),
)


@jax.jit
def kernel(x, edge_index, batch, W1, b1, W2, b2, W3, b3, Wl, bl):
    src = edge_index[0].astype(jnp.int32)
    dst = edge_index[1].astype(jnp.int32)
    pad = E_PAD - E
    src3 = jnp.concatenate([src, jnp.zeros((pad,), jnp.int32)]).reshape(
        NW, K, CHUNK
    )
    dst3 = jnp.concatenate([dst, jnp.full((pad,), DUMP, jnp.int32)]).reshape(
        NW, K, CHUNK
    )
    x_p = jnp.concatenate([x, jnp.zeros((NP - N, D), jnp.float32)])
    batch_p = jnp.concatenate(
        [batch.astype(jnp.int32), jnp.full((NP - N,), G + 7, jnp.int32)]
    ).reshape(NP, 1)
    zer_h = jnp.zeros((NP, H), jnp.float32)
    zer_d = jnp.zeros((NP, DEG_W), jnp.float32)
    ones_d = jnp.ones((CHUNK, DEG_W), jnp.float32)

    deg_kernel, agg_kernel = _sc_kernels()
    deg8 = deg_kernel(dst3, zer_d, ones_d)
    y1, dinv = _tc_first(deg8, x_p, W1)
    a1 = agg_kernel(y1, src3, dst3, zer_h)
    y2 = _tc_mid(a1, y1, dinv, b1.reshape(1, H), W2)
    a2 = agg_kernel(y2, src3, dst3, zer_h)
    y3 = _tc_mid(a2, y2, dinv, b2.reshape(1, H), W3)
    a3 = agg_kernel(y3, src3, dst3, zer_h)
    return _tc_final(a3, y3, dinv, b3.reshape(1, H), batch_p,
                     Wl, bl.reshape(1, O))


# speedup vs baseline: 13.6874x; 1.0702x over previous
import functools

import jax
import jax.numpy as jnp
from jax import lax
from jax.experimental import pallas as pl
from jax.experimental.pallas import tpu as pltpu
from jax.experimental.pallas import tpu_sc as plsc

N = 10000
E = 320000
D = 128
H = 64
G = 128
O = 2

NC = 2
NS = 16
NW = NC * NS
CHUNK = 128
K = 80
EPW = K * CHUNK
E_PAD = NW * EPW
NP = 10112
RPT = NP // NS
DUMP = 10104
DEG_W = 8


def _deg_body(dst3, zrows, ones_h, out, di_all, ones_v, stage, acc):
    c = lax.axis_index("c")
    s = lax.axis_index("s")
    wid = c * NS + s
    r0 = s * RPT
    pltpu.sync_copy(zrows.at[pl.ds(r0, RPT)], stage)
    pltpu.sync_copy(stage, acc.at[pl.ds(r0, RPT)])
    pltpu.sync_copy(ones_h, ones_v)
    pltpu.sync_copy(dst3.at[wid], di_all)
    plsc.subcore_barrier()

    def body(g, carry):
        pltpu.sync_copy(ones_v, acc.at[di_all.at[g]], add=True)
        return carry

    lax.fori_loop(0, K, body, 0)
    plsc.subcore_barrier()
    pltpu.sync_copy(acc.at[pl.ds(r0, RPT)], stage)
    pltpu.sync_copy(stage, out.at[c, pl.ds(r0, RPT)])


def _agg_body(y, src3, dst3, zrows, out, si_all, di_all, rows, stage, acc,
              gs0, gs1, gs2, gs3, ss0, ss1, ss2, ss3):
    c = lax.axis_index("c")
    s = lax.axis_index("s")
    wid = c * NS + s
    r0 = s * RPT
    pltpu.sync_copy(zrows.at[pl.ds(r0, 320)], stage)
    pltpu.sync_copy(stage, acc.at[pl.ds(r0, 320)])
    pltpu.sync_copy(zrows.at[pl.ds(r0 + 320, 312)], stage.at[pl.ds(0, 312)])
    pltpu.sync_copy(stage.at[pl.ds(0, 312)], acc.at[pl.ds(r0 + 320, 312)])
    pltpu.sync_copy(src3.at[wid], si_all)
    pltpu.sync_copy(dst3.at[wid], di_all)
    plsc.subcore_barrier()

    gsem = (gs0, gs1, gs2, gs3)
    ssem = (ss0, ss1, ss2, ss3)

    def g_start(g, b):
        pltpu.async_copy(y.at[si_all.at[g]], rows.at[b], gsem[b])

    def g_wait(g, b):
        pltpu.make_async_copy(y.at[si_all.at[g]], rows.at[b], gsem[b]).wait()

    def s_start(g, b):
        pltpu.async_copy(rows.at[b], acc.at[di_all.at[g]], ssem[b], add=True)

    def s_wait(g, b):
        pltpu.make_async_copy(rows.at[b], acc.at[di_all.at[g]], ssem[b]).wait()

    g_start(0, 0)
    g_start(1, 1)

    def body(i, carry):
        for b in range(4):
            t = 4 * i + b
            g_wait(t, b)
            s_start(t, b)

            @pl.when(t >= 2)
            def _():
                s_wait(t - 2, (b - 2) % 4)

            @pl.when(t + 2 < K)
            def _():
                g_start(t + 2, (b + 2) % 4)
        return carry

    lax.fori_loop(0, K // 4, body, 0)
    s_wait(K - 2, (K - 2) % 4)
    s_wait(K - 1, (K - 1) % 4)
    plsc.subcore_barrier()
    pltpu.sync_copy(acc.at[pl.ds(r0, 320)], stage)
    pltpu.sync_copy(stage, out.at[c, pl.ds(r0, 320)])
    pltpu.sync_copy(acc.at[pl.ds(r0 + 320, 312)], stage.at[pl.ds(0, 312)])
    pltpu.sync_copy(stage.at[pl.ds(0, 312)], out.at[c, pl.ds(r0 + 320, 312)])


@functools.cache
def _sc_kernels():
    mesh = plsc.VectorSubcoreMesh(
        core_axis_name="c", subcore_axis_name="s",
        num_cores=NC, num_subcores=NS,
    )
    params = pltpu.CompilerParams(use_tc_tiling_on_sc=False)
    deg = pl.kernel(
        _deg_body,
        out_type=jax.ShapeDtypeStruct((NC, NP, DEG_W), jnp.float32),
        mesh=mesh,
        compiler_params=params,
        scratch_types=[
            pltpu.VMEM((K, CHUNK), jnp.int32),
            pltpu.VMEM((CHUNK, DEG_W), jnp.float32),
            pltpu.VMEM((RPT, DEG_W), jnp.float32),
            pltpu.VMEM_SHARED((NP, DEG_W), jnp.float32),
        ],
    )
    agg = pl.kernel(
        _agg_body,
        out_type=jax.ShapeDtypeStruct((NC, NP, H), jnp.float32),
        mesh=mesh,
        compiler_params=params,
        scratch_types=[
            pltpu.VMEM((K, CHUNK), jnp.int32),
            pltpu.VMEM((K, CHUNK), jnp.int32),
            pltpu.VMEM((4, CHUNK, H), jnp.float32),
            pltpu.VMEM((320, H), jnp.float32),
            pltpu.VMEM_SHARED((NP, H), jnp.float32),
        ] + [pltpu.SemaphoreType.DMA] * 8,
    )
    return deg, agg


def _tc_first_body(deg8_ref, x_ref, w_ref, y_ref, dinv_ref):
    deg = deg8_ref[0, :, 0:1] + deg8_ref[1, :, 0:1] + 1.0
    dinv = lax.rsqrt(deg)
    xw = jnp.dot(x_ref[...], w_ref[...], preferred_element_type=jnp.float32)
    y_ref[...] = xw * dinv
    dinv_ref[...] = dinv


_tc_first = pl.pallas_call(
    _tc_first_body,
    out_shape=(
        jax.ShapeDtypeStruct((NP, H), jnp.float32),
        jax.ShapeDtypeStruct((NP, 1), jnp.float32),
    ),
)


def _tc_mid_body(agg_ref, y_ref, dinv_ref, b_ref, w_ref, out_ref):
    dinv = dinv_ref[...]
    tot = agg_ref[0] + agg_ref[1] + y_ref[...]
    h = jnp.maximum(tot * dinv + b_ref[...], 0.0)
    out_ref[...] = (
        jnp.dot(h, w_ref[...], preferred_element_type=jnp.float32) * dinv
    )


_tc_mid = pl.pallas_call(
    _tc_mid_body,
    out_shape=jax.ShapeDtypeStruct((NP, H), jnp.float32),
)


def _tc_final_body(agg_ref, y_ref, dinv_ref, b_ref, batch_ref, wl_ref, bl_ref,
                   out_ref):
    dinv = dinv_ref[...]
    tot = agg_ref[0] + agg_ref[1] + y_ref[...]
    h = jnp.maximum(tot * dinv + b_ref[...], 0.0)
    gid = lax.broadcasted_iota(jnp.int32, (NP, G), 1)
    oh = (batch_ref[...] == gid).astype(jnp.float32)
    dn = (((0,), (0,)), ((), ()))
    sums = lax.dot_general(oh, h, dn, preferred_element_type=jnp.float32)
    counts = lax.dot_general(
        oh, jnp.ones((NP, 1), jnp.float32), dn,
        preferred_element_type=jnp.float32,
    )
    pooled = sums / jnp.maximum(counts, 1.0)
    out_ref[...] = (
        jnp.dot(pooled, wl_ref[...], preferred_element_type=jnp.float32)
        + bl_ref[...]
    )


_tc_final = pl.pallas_call(
    _tc_final_body,
    out_shape=jax.ShapeDtypeStruct((G, O), jnp.float32),
)


@jax.jit
def kernel(x, edge_index, batch, W1, b1, W2, b2, W3, b3, Wl, bl):
    src = edge_index[0].astype(jnp.int32)
    dst = edge_index[1].astype(jnp.int32)
    pad = E_PAD - E
    src3 = jnp.concatenate([src, jnp.zeros((pad,), jnp.int32)]).reshape(
        NW, K, CHUNK
    )
    dst3 = jnp.concatenate([dst, jnp.full((pad,), DUMP, jnp.int32)]).reshape(
        NW, K, CHUNK
    )
    x_p = jnp.concatenate([x, jnp.zeros((NP - N, D), jnp.float32)])
    batch_p = jnp.concatenate(
        [batch.astype(jnp.int32), jnp.full((NP - N,), G + 7, jnp.int32)]
    ).reshape(NP, 1)
    zer_h = jnp.zeros((NP, H), jnp.float32)
    zer_d = jnp.zeros((NP, DEG_W), jnp.float32)
    ones_d = jnp.ones((CHUNK, DEG_W), jnp.float32)

    deg_kernel, agg_kernel = _sc_kernels()
    deg8 = deg_kernel(dst3, zer_d, ones_d)
    y1, dinv = _tc_first(deg8, x_p, W1)
    a1 = agg_kernel(y1, src3, dst3, zer_h)
    y2 = _tc_mid(a1, y1, dinv, b1.reshape(1, H), W2)
    a2 = agg_kernel(y2, src3, dst3, zer_h)
    y3 = _tc_mid(a2, y2, dinv, b2.reshape(1, H), W3)
    a3 = agg_kernel(y3, src3, dst3, zer_h)
    return _tc_final(a3, y3, dinv, b3.reshape(1, H), batch_p,
                     Wl, bl.reshape(1, O))


# speedup vs baseline: 33.4218x; 2.4418x over previous
import functools

import jax
import jax.numpy as jnp
from jax import lax
from jax.experimental import pallas as pl
from jax.experimental.pallas import tpu as pltpu
from jax.experimental.pallas import tpu_sc as plsc

N = 10000
E = 320000
D = 128
H = 64
G = 128
O = 2

NC = 2
NS = 16
NW = NC * NS
CHUNK = 128
K = 80
EPW = K * CHUNK
E_PAD = NW * EPW
NP = 10112
RPT = NP // NS
DUMP = 10104
DEG_W = 8


def _deg_body(dst3, zrows, ones_h, out, di_all, ones_v, stage, acc):
    c = lax.axis_index("c")
    s = lax.axis_index("s")
    wid = c * NS + s
    r0 = s * RPT
    pltpu.sync_copy(zrows.at[pl.ds(r0, RPT)], stage)
    pltpu.sync_copy(stage, acc.at[pl.ds(r0, RPT)])
    pltpu.sync_copy(ones_h, ones_v)
    pltpu.sync_copy(dst3.at[wid], di_all)
    plsc.subcore_barrier()

    def body(g, carry):
        pltpu.sync_copy(ones_v, acc.at[di_all.at[g]], add=True)
        return carry

    lax.fori_loop(0, K, body, 0)
    plsc.subcore_barrier()
    pltpu.sync_copy(acc.at[pl.ds(r0, RPT)], stage)
    pltpu.sync_copy(stage, out.at[c, pl.ds(r0, RPT)])


def _agg_body(y, src3, dst3, out, si_all, di_all, rows, acc, ysh,
              gs0, gs1, gs2, gs3, ss0, ss1, ss2, ss3):
    c = lax.axis_index("c")
    s = lax.axis_index("s")
    wid = c * NS + s
    r0 = s * RPT
    stg = rows
    pltpu.sync_copy(y.at[pl.ds(r0, 320)], stg.at[pl.ds(0, 320)])
    pltpu.sync_copy(stg.at[pl.ds(0, 320)], ysh.at[pl.ds(r0, 320)])
    pltpu.sync_copy(y.at[pl.ds(r0 + 320, 312)], stg.at[pl.ds(0, 312)])
    pltpu.sync_copy(stg.at[pl.ds(0, 312)], ysh.at[pl.ds(r0 + 320, 312)])

    def zrow(i, carry):
        for j in range(H // 16):
            stg[i, pl.ds(j * 16, 16)] = jnp.zeros((16,), jnp.float32)
        return carry

    lax.fori_loop(0, 320, zrow, 0)
    pltpu.sync_copy(stg.at[pl.ds(0, 320)], acc.at[pl.ds(r0, 320)])
    pltpu.sync_copy(stg.at[pl.ds(0, 312)], acc.at[pl.ds(r0 + 320, 312)])
    plsc.subcore_barrier()

    gsem = (gs0, gs1, gs2, gs3)
    ssem = (ss0, ss1, ss2, ss3)
    KH = K // 2

    def g_start(g, b):
        pltpu.async_copy(
            ysh.at[si_all.at[g]], rows.at[pl.ds(b * CHUNK, CHUNK)], gsem[b]
        )

    def g_wait(g, b):
        pltpu.make_async_copy(
            ysh.at[si_all.at[g]], rows.at[pl.ds(b * CHUNK, CHUNK)], gsem[b]
        ).wait()

    def s_start(g, b):
        pltpu.async_copy(
            rows.at[pl.ds(b * CHUNK, CHUNK)], acc.at[di_all.at[g]], ssem[b],
            add=True,
        )

    def s_wait(g, b):
        pltpu.make_async_copy(
            rows.at[pl.ds(b * CHUNK, CHUNK)], acc.at[di_all.at[g]], ssem[b]
        ).wait()

    for h in range(2):
        pltpu.sync_copy(src3.at[wid, pl.ds(h * KH, KH)], si_all)
        pltpu.sync_copy(dst3.at[wid, pl.ds(h * KH, KH)], di_all)
        g_start(0, 0)
        g_start(1, 1)

        def body(i, carry):
            for b in range(4):
                t = 4 * i + b
                g_wait(t, b)
                s_start(t, b)

                @pl.when(t >= 2)
                def _():
                    s_wait(t - 2, (b - 2) % 4)

                @pl.when(t + 2 < KH)
                def _():
                    g_start(t + 2, (b + 2) % 4)
            return carry

        lax.fori_loop(0, KH // 4, body, 0)
        s_wait(KH - 2, (KH - 2) % 4)
        s_wait(KH - 1, (KH - 1) % 4)

    plsc.subcore_barrier()
    pltpu.sync_copy(acc.at[pl.ds(r0, 320)], stg.at[pl.ds(0, 320)])
    pltpu.sync_copy(stg.at[pl.ds(0, 320)], out.at[c, pl.ds(r0, 320)])
    pltpu.sync_copy(acc.at[pl.ds(r0 + 320, 312)], stg.at[pl.ds(0, 312)])
    pltpu.sync_copy(stg.at[pl.ds(0, 312)], out.at[c, pl.ds(r0 + 320, 312)])


@functools.cache
def _sc_kernels():
    mesh = plsc.VectorSubcoreMesh(
        core_axis_name="c", subcore_axis_name="s",
        num_cores=NC, num_subcores=NS,
    )
    params = pltpu.CompilerParams(use_tc_tiling_on_sc=False)
    deg = pl.kernel(
        _deg_body,
        out_type=jax.ShapeDtypeStruct((NC, NP, DEG_W), jnp.float32),
        mesh=mesh,
        compiler_params=params,
        scratch_types=[
            pltpu.VMEM((K, CHUNK), jnp.int32),
            pltpu.VMEM((CHUNK, DEG_W), jnp.float32),
            pltpu.VMEM((RPT, DEG_W), jnp.float32),
            pltpu.VMEM_SHARED((NP, DEG_W), jnp.float32),
        ],
    )
    agg = pl.kernel(
        _agg_body,
        out_type=jax.ShapeDtypeStruct((NC, NP, H), jnp.float32),
        mesh=mesh,
        compiler_params=params,
        scratch_types=[
            pltpu.VMEM((K // 2, CHUNK), jnp.int32),
            pltpu.VMEM((K // 2, CHUNK), jnp.int32),
            pltpu.VMEM((4 * CHUNK, H), jnp.float32),
            pltpu.VMEM_SHARED((NP, H), jnp.float32),
            pltpu.VMEM_SHARED((NP, H), jnp.float32),
        ] + [pltpu.SemaphoreType.DMA] * 8,
    )
    return deg, agg


def _tc_first_body(deg8_ref, x_ref, w_ref, y_ref, dinv_ref):
    deg = deg8_ref[0, :, 0:1] + deg8_ref[1, :, 0:1] + 1.0
    dinv = lax.rsqrt(deg)
    xw = jnp.dot(x_ref[...], w_ref[...], preferred_element_type=jnp.float32)
    y_ref[...] = xw * dinv
    dinv_ref[...] = dinv


_tc_first = pl.pallas_call(
    _tc_first_body,
    out_shape=(
        jax.ShapeDtypeStruct((NP, H), jnp.float32),
        jax.ShapeDtypeStruct((NP, 1), jnp.float32),
    ),
)


def _tc_mid_body(agg_ref, y_ref, dinv_ref, b_ref, w_ref, out_ref):
    dinv = dinv_ref[...]
    tot = agg_ref[0] + agg_ref[1] + y_ref[...]
    h = jnp.maximum(tot * dinv + b_ref[...], 0.0)
    out_ref[...] = (
        jnp.dot(h, w_ref[...], preferred_element_type=jnp.float32) * dinv
    )


_tc_mid = pl.pallas_call(
    _tc_mid_body,
    out_shape=jax.ShapeDtypeStruct((NP, H), jnp.float32),
)


def _tc_final_body(agg_ref, y_ref, dinv_ref, b_ref, batch_ref, wl_ref, bl_ref,
                   out_ref):
    dinv = dinv_ref[...]
    tot = agg_ref[0] + agg_ref[1] + y_ref[...]
    h = jnp.maximum(tot * dinv + b_ref[...], 0.0)
    gid = lax.broadcasted_iota(jnp.int32, (NP, G), 1)
    oh = (batch_ref[...] == gid).astype(jnp.float32)
    dn = (((0,), (0,)), ((), ()))
    sums = lax.dot_general(oh, h, dn, preferred_element_type=jnp.float32)
    counts = lax.dot_general(
        oh, jnp.ones((NP, 1), jnp.float32), dn,
        preferred_element_type=jnp.float32,
    )
    pooled = sums / jnp.maximum(counts, 1.0)
    out_ref[...] = (
        jnp.dot(pooled, wl_ref[...], preferred_element_type=jnp.float32)
        + bl_ref[...]
    )


_tc_final = pl.pallas_call(
    _tc_final_body,
    out_shape=jax.ShapeDtypeStruct((G, O), jnp.float32),
)


@jax.jit
def kernel(x, edge_index, batch, W1, b1, W2, b2, W3, b3, Wl, bl):
    src = edge_index[0].astype(jnp.int32)
    dst = edge_index[1].astype(jnp.int32)
    pad = E_PAD - E
    src3 = jnp.concatenate([src, jnp.zeros((pad,), jnp.int32)]).reshape(
        NW, K, CHUNK
    )
    dst3 = jnp.concatenate([dst, jnp.full((pad,), DUMP, jnp.int32)]).reshape(
        NW, K, CHUNK
    )
    x_p = jnp.concatenate([x, jnp.zeros((NP - N, D), jnp.float32)])
    batch_p = jnp.concatenate(
        [batch.astype(jnp.int32), jnp.full((NP - N,), G + 7, jnp.int32)]
    ).reshape(NP, 1)
    zer_d = jnp.zeros((NP, DEG_W), jnp.float32)
    ones_d = jnp.ones((CHUNK, DEG_W), jnp.float32)

    deg_kernel, agg_kernel = _sc_kernels()
    deg8 = deg_kernel(dst3, zer_d, ones_d)
    y1, dinv = _tc_first(deg8, x_p, W1)
    a1 = agg_kernel(y1, src3, dst3)
    y2 = _tc_mid(a1, y1, dinv, b1.reshape(1, H), W2)
    a2 = agg_kernel(y2, src3, dst3)
    y3 = _tc_mid(a2, y2, dinv, b2.reshape(1, H), W3)
    a3 = agg_kernel(y3, src3, dst3)
    return _tc_final(a3, y3, dinv, b3.reshape(1, H), batch_p,
                     Wl, bl.reshape(1, O))
